# Initial kernel scaffold; baseline (speedup 1.0000x reference)
import jax
import jax.numpy as jnp
from jax.experimental import pallas as pl


def kernel(x, target):
    raise NotImplementedError("write your pallas kernel here")


# speedup vs baseline: 7.7305x; 7.7305x over previous
import jax
import jax.numpy as jnp
import numpy as np
from jax.experimental import pallas as pl
from jax.experimental.pallas import tpu as pltpu

_SIZE = 32768
_PAD = 0
_SMOOTH = 0.1
_CONF = 1.0 - _SMOOTH
_EPS = _SMOOTH / (_SIZE - 2)
_K = _CONF * float(np.log(_CONF)) + _SMOOTH * float(np.log(_EPS))

_BR = 256
_BC = 8192


def _loss_kernel(n_rows, tgt_ref, x_ref, out_ref, acc_ref):
    i = pl.program_id(0)
    j = pl.program_id(1)
    nr = pl.num_programs(0)
    nc = pl.num_programs(1)

    @pl.when((i == 0) & (j == 0))
    def _init():
        acc_ref[0] = 0.0
        acc_ref[1] = 0.0

    tgt = tgt_ref[0]
    nonpad = tgt != _PAD
    x = x_ref[...]
    cols = jax.lax.broadcasted_iota(jnp.int32, (_BR, _BC), 1) + j * _BC
    w = jnp.where(cols == tgt, _CONF, _EPS)
    w = jnp.where(cols == _PAD, 0.0, w)
    w = jnp.where(nonpad, w, 0.0)
    acc_ref[0] += jnp.sum(w * x)

    @pl.when(j == 0)
    def _count():
        acc_ref[1] += jnp.sum(nonpad.astype(jnp.float32))

    @pl.when((i == nr - 1) & (j == nc - 1))
    def _finish():
        out_ref[0, 0] = (acc_ref[1] * _K - acc_ref[0]) / n_rows


def kernel(x, target):
    n, size = x.shape
    nr = n // _BR
    nc = size // _BC
    tgt3 = target.astype(jnp.int32).reshape(nr, _BR, 1)
    import functools
    out = pl.pallas_call(
        functools.partial(_loss_kernel, float(n)),
        grid=(nr, nc),
        in_specs=[
            pl.BlockSpec((1, _BR, 1), lambda i, j: (i, 0, 0)),
            pl.BlockSpec((_BR, _BC), lambda i, j: (i, j)),
        ],
        out_specs=pl.BlockSpec(memory_space=pltpu.SMEM),
        out_shape=jax.ShapeDtypeStruct((1, 1), jnp.float32),
        scratch_shapes=[pltpu.SMEM((2,), jnp.float32)],
    )(tgt3, x)
    return out[0, 0]


# speedup vs baseline: 8.4118x; 1.0881x over previous
import jax
import jax.numpy as jnp
import numpy as np
from jax.experimental import pallas as pl
from jax.experimental.pallas import tpu as pltpu

_SIZE = 32768
_PAD = 0
_SMOOTH = 0.1
_CONF = 1.0 - _SMOOTH
_EPS = _SMOOTH / (_SIZE - 2)
_K = _CONF * float(np.log(_CONF)) + _SMOOTH * float(np.log(_EPS))

_BR = 512
_BC = 8192


def _loss_kernel(n_rows, tgt_ref, x_ref, out_ref, acc_ref):
    i = pl.program_id(0)
    j = pl.program_id(1)
    nr = pl.num_programs(0)
    nc = pl.num_programs(1)

    @pl.when((i == 0) & (j == 0))
    def _init():
        acc_ref[0] = 0.0
        acc_ref[1] = 0.0

    tgt = tgt_ref[0]
    nonpad = tgt != _PAD
    x = x_ref[...]
    cols = jax.lax.broadcasted_iota(jnp.int32, (_BR, _BC), 1) + j * _BC
    w = jnp.where(cols == tgt, _CONF, _EPS)
    w = jnp.where(cols == _PAD, 0.0, w)
    w = jnp.where(nonpad, w, 0.0)
    acc_ref[0] += jnp.sum(w * x)

    @pl.when(j == 0)
    def _count():
        acc_ref[1] += jnp.sum(nonpad.astype(jnp.float32))

    @pl.when((i == nr - 1) & (j == nc - 1))
    def _finish():
        out_ref[0, 0] = (acc_ref[1] * _K - acc_ref[0]) / n_rows


def kernel(x, target):
    n, size = x.shape
    nr = n // _BR
    nc = size // _BC
    tgt3 = target.astype(jnp.int32).reshape(nr, _BR, 1)
    import functools
    out = pl.pallas_call(
        functools.partial(_loss_kernel, float(n)),
        grid=(nr, nc),
        in_specs=[
            pl.BlockSpec((1, _BR, 1), lambda i, j: (i, 0, 0)),
            pl.BlockSpec((_BR, _BC), lambda i, j: (i, j)),
        ],
        out_specs=pl.BlockSpec(memory_space=pltpu.SMEM),
        out_shape=jax.ShapeDtypeStruct((1, 1), jnp.float32),
        scratch_shapes=[pltpu.SMEM((2,), jnp.float32)],
    )(tgt3, x)
    return out[0, 0]


# speedup vs baseline: 8.7084x; 1.0353x over previous
import jax
import jax.numpy as jnp
import numpy as np
from jax.experimental import pallas as pl
from jax.experimental.pallas import tpu as pltpu

_SIZE = 32768
_PAD = 0
_SMOOTH = 0.1
_CONF = 1.0 - _SMOOTH
_EPS = _SMOOTH / (_SIZE - 2)
_K = _CONF * float(np.log(_CONF)) + _SMOOTH * float(np.log(_EPS))

_BR = 128
_BC = 32768


def _loss_kernel(n_rows, tgt_ref, x_ref, out_ref, acc_ref):
    i = pl.program_id(0)
    j = pl.program_id(1)
    nr = pl.num_programs(0)
    nc = pl.num_programs(1)

    @pl.when((i == 0) & (j == 0))
    def _init():
        acc_ref[0] = 0.0
        acc_ref[1] = 0.0

    tgt = tgt_ref[0]
    nonpad = tgt != _PAD
    x = x_ref[...]
    cols = jax.lax.broadcasted_iota(jnp.int32, (_BR, _BC), 1) + j * _BC
    w = jnp.where(cols == tgt, _CONF, _EPS)
    w = jnp.where(cols == _PAD, 0.0, w)
    w = jnp.where(nonpad, w, 0.0)
    acc_ref[0] += jnp.sum(w * x)

    @pl.when(j == 0)
    def _count():
        acc_ref[1] += jnp.sum(nonpad.astype(jnp.float32))

    @pl.when((i == nr - 1) & (j == nc - 1))
    def _finish():
        out_ref[0, 0] = (acc_ref[1] * _K - acc_ref[0]) / n_rows


def kernel(x, target):
    n, size = x.shape
    nr = n // _BR
    nc = size // _BC
    tgt3 = target.astype(jnp.int32).reshape(nr, _BR, 1)
    import functools
    out = pl.pallas_call(
        functools.partial(_loss_kernel, float(n)),
        grid=(nr, nc),
        in_specs=[
            pl.BlockSpec((1, _BR, 1), lambda i, j: (i, 0, 0)),
            pl.BlockSpec((_BR, _BC), lambda i, j: (i, j)),
        ],
        out_specs=pl.BlockSpec(memory_space=pltpu.SMEM),
        out_shape=jax.ShapeDtypeStruct((1, 1), jnp.float32),
        scratch_shapes=[pltpu.SMEM((2,), jnp.float32)],
    )(tgt3, x)
    return out[0, 0]
